# baseline (speedup 1.0000x reference)
import jax
import jax.numpy as jnp
from jax.experimental import pallas as pl
from jax.experimental.pallas import tpu as pltpu

ALPHA = 0.1


def _mb_kernel(idx_ref, hid_ref, mem_ref, out_ref, sums_ref, cnt_ref, iota_ref):
    i = pl.program_id(0)
    nsteps = pl.num_programs(0)
    K, TB = idx_ref.shape
    N = mem_ref.shape[0]

    @pl.when(i == 0)
    def _init():
        sums_ref[...] = jnp.zeros_like(sums_ref)
        cnt_ref[...] = jnp.zeros_like(cnt_ref)
        iota_ref[...] = jax.lax.broadcasted_iota(jnp.int32, iota_ref.shape, 0)

    idx = idx_ref[...]
    n_iota = iota_ref[...]
    one_bits = jnp.int32(0x3F800000)
    acc = jnp.where(idx[0:1, :] == n_iota, one_bits, 0)
    for k in range(1, K):
        acc = jnp.where(idx[k : k + 1, :] == n_iota, one_bits, acc)
    maskf = jax.lax.bitcast_convert_type(acc, jnp.float32)

    cnt_ref[...] += jnp.sum(maskf, axis=1, keepdims=True)
    sums_ref[...] += jax.lax.dot_general(
        maskf, hid_ref[...], (((1,), (0,)), ((), ())),
        preferred_element_type=jnp.float32)

    @pl.when(i == nsteps - 1)
    def _finish():
        counts = cnt_ref[...]
        agg = sums_ref[...] / jnp.maximum(counts, 1.0)
        memf = mem_ref[...].astype(jnp.float32)
        upd = ALPHA * agg + (1.0 - ALPHA) * memf
        out_ref[...] = jnp.where(counts > 0.0, upd, memf).astype(jnp.bfloat16)


def kernel(hidden_states, batch_idx, slot_indices, memory):
    T, D = hidden_states.shape
    K = slot_indices.shape[1]
    N = memory.shape[1]
    TB = 1024
    idx_t = slot_indices.T.astype(jnp.int32)
    mem2d = memory[0]
    out = pl.pallas_call(
        _mb_kernel,
        grid=(T // TB,),
        in_specs=[
            pl.BlockSpec((K, TB), lambda i: (0, i)),
            pl.BlockSpec((TB, D), lambda i: (i, 0)),
            pl.BlockSpec((N, D), lambda i: (0, 0)),
        ],
        out_specs=pl.BlockSpec((N, D), lambda i: (0, 0)),
        out_shape=jax.ShapeDtypeStruct((N, D), jnp.bfloat16),
        scratch_shapes=[
            pltpu.VMEM((N, D), jnp.float32),
            pltpu.VMEM((N, 1), jnp.float32),
            pltpu.VMEM((N, TB), jnp.int32),
        ],
    )(idx_t, hidden_states, mem2d)
    return out[None]


# speedup vs baseline: 1.1024x; 1.1024x over previous
import jax
import jax.numpy as jnp
from jax.experimental import pallas as pl
from jax.experimental.pallas import tpu as pltpu

ALPHA = 0.1


def _mb_kernel(idx_ref, hid_ref, mem_ref, out_ref, sums_ref, cnt_ref, iota_ref):
    i = pl.program_id(0)
    nsteps = pl.num_programs(0)
    K, TB = idx_ref.shape
    N = mem_ref.shape[0]

    @pl.when(i == 0)
    def _init():
        sums_ref[...] = jnp.zeros_like(sums_ref)
        cnt_ref[...] = jnp.zeros_like(cnt_ref)
        iota_ref[...] = jax.lax.broadcasted_iota(jnp.int32, iota_ref.shape, 0)

    idx = idx_ref[...]
    W = N // 32
    one_bits = jnp.int32(0x3F800000)

    w_iota = iota_ref[...]
    bits = jnp.zeros((W, TB), jnp.int32)
    for k in range(K):
        col = idx[k : k + 1, :]
        bitk = jnp.left_shift(jnp.int32(1), col & 31)
        bits = bits | jnp.where((col >> 5) == w_iota, bitk, 0)

    sub_iota = jax.lax.broadcasted_iota(jnp.int32, (32, 1), 0)
    pattern = jnp.left_shift(jnp.int32(1), sub_iota)
    groups = []
    for g in range(W):
        hit = (bits[g : g + 1, :] & pattern) != 0
        groups.append(jnp.where(hit, one_bits, 0))
    maskf = jax.lax.bitcast_convert_type(
        jnp.concatenate(groups, axis=0), jnp.float32)

    cnt_ref[...] += jnp.sum(maskf, axis=1, keepdims=True)
    sums_ref[...] += jax.lax.dot_general(
        maskf, hid_ref[...], (((1,), (0,)), ((), ())),
        preferred_element_type=jnp.float32)

    @pl.when(i == nsteps - 1)
    def _finish():
        counts = cnt_ref[...]
        agg = sums_ref[...] / jnp.maximum(counts, 1.0)
        memf = mem_ref[...].astype(jnp.float32)
        upd = ALPHA * agg + (1.0 - ALPHA) * memf
        out_ref[...] = jnp.where(counts > 0.0, upd, memf).astype(jnp.bfloat16)


def kernel(hidden_states, batch_idx, slot_indices, memory):
    T, D = hidden_states.shape
    K = slot_indices.shape[1]
    N = memory.shape[1]
    TB = 1024
    idx_t = slot_indices.T.astype(jnp.int32)
    mem2d = memory[0]
    out = pl.pallas_call(
        _mb_kernel,
        grid=(T // TB,),
        in_specs=[
            pl.BlockSpec((K, TB), lambda i: (0, i)),
            pl.BlockSpec((TB, D), lambda i: (i, 0)),
            pl.BlockSpec((N, D), lambda i: (0, 0)),
        ],
        out_specs=pl.BlockSpec((N, D), lambda i: (0, 0)),
        out_shape=jax.ShapeDtypeStruct((N, D), jnp.bfloat16),
        scratch_shapes=[
            pltpu.VMEM((N, D), jnp.float32),
            pltpu.VMEM((N, 1), jnp.float32),
            pltpu.VMEM((N // 32, TB), jnp.int32),
        ],
    )(idx_t, hidden_states, mem2d)
    return out[None]
